# Initial kernel scaffold; baseline (speedup 1.0000x reference)
import functools

import jax
import jax.numpy as jnp
from jax import lax
from jax.experimental import pallas as pl
from jax.experimental.pallas import tpu as pltpu

B = 1024
N = 50000
NPAD = 50176
DIN = 128
D = 256
DB = 512
CTX = 96
KE = 4
EPS = 1e-5
RB = 1024
NBLK = NPAD // RB
QB = 16
BIG = jnp.float32(3e38)


def _ln(x, g, b):
    mu = jnp.mean(x, axis=-1, keepdims=True)
    var = jnp.mean((x - mu) ** 2, axis=-1, keepdims=True)
    return (x - mu) / jnp.sqrt(var + EPS) * g + b


def _mm(a, b):
    return lax.dot_general(a, b, (((1,), (0,)), ((), ())),
                           preferred_element_type=jnp.float32)


def _encode(cx, Wl, bl, W1, b1, W2, b2, g, bm, WK, bK):
    x = _mm(cx, Wl) + bl
    h = _mm(jax.nn.relu(_mm(x, W1) + b1), W2) + b2
    x = x + h
    k = _mm(_ln(x, g, bm), WK) + bK
    return x, k


def _bank_body(cx_ref, y_ref, Wl, bl, W1, b1, W2, b2, g, bm, WK, bK, out_ref):
    _, ck = _encode(cx_ref[...], Wl[...], bl[...], W1[...], b1[...],
                    W2[...], b2[...], g[...], bm[...], WK[...], bK[...])
    out_ref[:, :D] = ck
    out_ref[:, D:] = y_ref[...]


def _query_body(x_ref, Wl, bl, W1, b1, W2, b2, g, bm, WK, bK, x_out, k_out):
    x, k = _encode(x_ref[...], Wl[...], bl[...], W1[...], b1[...],
                   W2[...], b2[...], g[...], bm[...], WK[...], bK[...])
    x_out[...] = x
    k_out[...] = k


def _dist_body(k_ref, bank_ref, s_ref):
    j = pl.program_id(0)
    ck = bank_ref[:, :D]
    dot = lax.dot_general(k_ref[...], ck, (((1,), (1,)), ((), ())),
                          preferred_element_type=jnp.float32)
    cn = jnp.sum(ck * ck, axis=1)
    s = cn[None, :] - 2.0 * dot
    col = j * RB + lax.broadcasted_iota(jnp.int32, (1, RB), 1)
    s_ref[...] = jnp.where(col >= N, BIG, s)


def _ctx_body(x_ref, k_ref, ctx_ref, wlab, blab, Wt1, bt1, Wt2,
              plw, plb, Wp1, bp1, Wp2, bp2, gh, bh, Wh, bhd, out_ref):
    ck = ctx_ref[:, :, :D]
    yc = ctx_ref[:, :, D:D + 1]
    kq = k_ref[...]
    sim = 2.0 * jnp.sum(kq[:, None, :] * ck, axis=-1) - jnp.sum(ck * ck, axis=-1)
    m = jnp.max(sim, axis=-1, keepdims=True)
    p = jnp.exp(sim - m)
    p = p / jnp.sum(p, axis=-1, keepdims=True)

    diff = (kq[:, None, :] - ck).reshape(QB * CTX, D)
    t = _mm(jax.nn.relu(_mm(diff, Wt1[...]) + bt1[...]), Wt2[...])
    vals = yc * wlab[...][None] + blab[...][None] + t.reshape(QB, CTX, D)
    ctx_x = jnp.sum(p[:, :, None] * vals, axis=1)

    x2 = x_ref[...] + ctx_x
    X = jnp.broadcast_to(x2[:, None, :], (QB, KE, D))
    mu = jnp.mean(X, axis=-1, keepdims=True)
    var = jnp.mean((X - mu) ** 2, axis=-1, keepdims=True)
    h = (X - mu) / jnp.sqrt(var + EPS) * plw[...][None] + plb[...][None]
    h2 = h.reshape(QB * KE, D)
    h2 = _mm(jax.nn.relu(_mm(h2, Wp1[...]) + bp1[...]), Wp2[...]) + bp2[...]
    X = X + h2.reshape(QB, KE, D)
    o = _ln(X, gh[...][None], bh[...][None])
    o = jax.nn.relu(o).reshape(QB * KE, D)
    out_ref[...] = _mm(o, Wh[...]) + bhd[...]


def _full(shape):
    nd = len(shape)
    return pl.BlockSpec(shape, lambda *_: (0,) * nd)


def kernel(x_num, candidate_x_num, candidate_y, W_lin, b_lin, W_e1, b_e1,
           W_e2, b_e2, g_mix, b_mix, W_K, b_K, w_label, b_label, W_t1, b_t1,
           W_t2, pln_w, pln_b, W_p1, b_p1, W_p2, b_p2, g_hln, b_hln, W_head,
           b_head, is_train):
    f32 = jnp.float32
    cxp = jnp.zeros((NPAD, DIN), f32).at[:N].set(candidate_x_num)
    y128 = jnp.broadcast_to(
        jnp.zeros((NPAD,), f32).at[:N].set(candidate_y)[:, None], (NPAD, DIN))
    bl = b_lin.reshape(1, D)
    b1 = b_e1.reshape(1, DB)
    b2 = b_e2.reshape(1, D)
    g = g_mix.reshape(1, D)
    bm = b_mix.reshape(1, D)
    bK = b_K.reshape(1, D)

    wspecs = [_full(W_lin.shape), _full((1, D)), _full(W_e1.shape),
              _full((1, DB)), _full(W_e2.shape), _full((1, D)),
              _full((1, D)), _full((1, D)), _full(W_K.shape), _full((1, D))]
    wargs = (W_lin, bl, W_e1, b1, W_e2, b2, g, bm, W_K, bK)

    bank = pl.pallas_call(
        _bank_body,
        grid=(NBLK,),
        in_specs=[pl.BlockSpec((RB, DIN), lambda j: (j, 0)),
                  pl.BlockSpec((RB, DIN), lambda j: (j, 0))] + wspecs,
        out_specs=pl.BlockSpec((RB, D + DIN), lambda j: (j, 0)),
        out_shape=jax.ShapeDtypeStruct((NPAD, D + DIN), f32),
    )(cxp, y128, *wargs)

    xq, kq = pl.pallas_call(
        _query_body,
        in_specs=[_full((B, DIN))] + wspecs,
        out_specs=[_full((B, D)), _full((B, D))],
        out_shape=[jax.ShapeDtypeStruct((B, D), f32),
                   jax.ShapeDtypeStruct((B, D), f32)],
    )(x_num, *wargs)

    s = pl.pallas_call(
        _dist_body,
        grid=(NBLK,),
        in_specs=[_full((B, D)), pl.BlockSpec((RB, D + DIN), lambda j: (j, 0))],
        out_specs=pl.BlockSpec((B, RB), lambda j: (0, j)),
        out_shape=jax.ShapeDtypeStruct((B, NPAD), f32),
    )(kq, bank)

    _, idx = lax.top_k(-s, CTX)
    ctx = bank[idx]

    out = pl.pallas_call(
        _ctx_body,
        grid=(B // QB,),
        in_specs=[pl.BlockSpec((QB, D), lambda j: (j, 0)),
                  pl.BlockSpec((QB, D), lambda j: (j, 0)),
                  pl.BlockSpec((QB, CTX, D + DIN), lambda j: (j, 0, 0)),
                  _full((1, D)), _full((1, D)), _full(W_t1.shape),
                  _full((1, DB)), _full(W_t2.shape), _full(pln_w.shape),
                  _full(pln_b.shape), _full(W_p1.shape), _full((1, DB)),
                  _full(W_p2.shape), _full((1, D)), _full((1, D)),
                  _full((1, D)), _full(W_head.shape), _full((1, 1))],
        out_specs=pl.BlockSpec((QB * KE, 1), lambda j: (j, 0)),
        out_shape=jax.ShapeDtypeStruct((B * KE, 1), f32),
    )(xq, kq, ctx, w_label.reshape(1, D), b_label.reshape(1, D), W_t1,
      b_t1.reshape(1, DB), W_t2, pln_w, pln_b, W_p1, b_p1.reshape(1, DB),
      W_p2, b_p2.reshape(1, D), g_hln.reshape(1, D), b_hln.reshape(1, D),
      W_head, b_head.reshape(1, 1))

    return out.reshape(B, KE, 1)


# speedup vs baseline: 5.0195x; 5.0195x over previous
import functools

import jax
import jax.numpy as jnp
from jax import lax
from jax.experimental import pallas as pl
from jax.experimental.pallas import tpu as pltpu
from jax.experimental.pallas import tpu_sc as plsc

B = 1024
N = 50000
NPAD = 50176
DIN = 128
D = 256
DB = 512
CTX = 96
KE = 4
EPS = 1e-5
RB = 1024
NBLK = NPAD // RB
QB = 16
BANKW = D + DIN
IMAX = 2147483647
PADORD = 0x7F000000

NW = 32
RPW = B // NW
NV = NPAD // 16
CAP = 8192
HIGHEST = lax.Precision.DEFAULT


def _ln(x, g, b):
    mu = jnp.mean(x, axis=-1, keepdims=True)
    var = jnp.mean((x - mu) ** 2, axis=-1, keepdims=True)
    return (x - mu) / jnp.sqrt(var + EPS) * g + b


def _mm(a, b):
    return lax.dot_general(a, b, (((1,), (0,)), ((), ())),
                           preferred_element_type=jnp.float32,
                           precision=HIGHEST)


def _encode(cx, Wl, bl, W1, b1, W2, b2, g, bm, WK, bK):
    x = _mm(cx, Wl) + bl
    h = _mm(jax.nn.relu(_mm(x, W1) + b1), W2) + b2
    x = x + h
    k = _mm(_ln(x, g, bm), WK) + bK
    return x, k


def _bank_body(cx_ref, y_ref, Wl, bl, W1, b1, W2, b2, g, bm, WK, bK, out_ref):
    _, ck = _encode(cx_ref[...], Wl[...], bl[...], W1[...], b1[...],
                    W2[...], b2[...], g[...], bm[...], WK[...], bK[...])
    out_ref[:, :D] = ck
    out_ref[:, D:] = y_ref[...]


def _query_body(x_ref, Wl, bl, W1, b1, W2, b2, g, bm, WK, bK, x_out, k_out):
    x, k = _encode(x_ref[...], Wl[...], bl[...], W1[...], b1[...],
                   W2[...], b2[...], g[...], bm[...], WK[...], bK[...])
    x_out[...] = x
    k_out[...] = k


def _dist_body(k_ref, bank_ref, o_ref):
    j = pl.program_id(0)
    ck = bank_ref[:, :D]
    kq = k_ref[...]
    dot = lax.dot_general(kq, ck, (((1,), (1,)), ((), ())),
                          preferred_element_type=jnp.float32,
                          precision=HIGHEST)
    cn = lax.dot_general(jnp.ones((1, D), jnp.float32), ck * ck,
                         (((1,), (1,)), ((), ())),
                         preferred_element_type=jnp.float32,
                         precision=lax.Precision.HIGHEST)
    kn = jnp.sum(kq * kq, axis=1, keepdims=True)
    s = (kn - 2.0 * dot) + cn
    bits = lax.bitcast_convert_type(s, jnp.int32)
    o = jnp.where(bits < 0, bits ^ 0x7FFFFFFF, bits)
    col = j * RB + lax.broadcasted_iota(jnp.int32, (1, RB), 1)
    o_ref[...] = jnp.where(col >= N, PADORD, o)


def _sc_body(ord_hbm, bank_hbm, ctx_hbm, srow, cord, cidx, oidx, ctxbuf, sem):
    wid = lax.axis_index("s") * 2 + lax.axis_index("c")
    iota = lax.iota(jnp.int32, 16)
    zeros16 = jnp.zeros((16,), jnp.int32)
    imaxv = jnp.full((16,), IMAX, jnp.int32)
    INTMIN = jnp.int32(-IMAX - 1)

    def _scal(v):
        return v[0]

    def row_body(rr, carry):
        r = wid * RPW + rr
        pltpu.sync_copy(ord_hbm.at[r], srow)

        def ga(g, accs):
            base = g * 96
            return tuple(jnp.minimum(accs[j], srow[pl.ds(base + j * 16, 16)])
                         for j in range(6))
        accs = lax.fori_loop(0, 522, ga, (imaxv,) * 6)
        for j in range(4):
            v = srow[pl.ds(522 * 96 + j * 16, 16)]
            accs = tuple(jnp.minimum(accs[i], v) if i == j else accs[i]
                         for i in range(6))
        m = accs[0]
        for j in range(1, 6):
            m = jnp.maximum(m, accs[j])
        U = m[0]
        for l in range(1, 16):
            U = jnp.maximum(U, m[l])

        def gb(i, c):
            for u in range(4):
                base = (i * 4 + u) * 16
                v = srow[pl.ds(base, 16)]
                msk = v <= U
                mi = msk.astype(jnp.int32)
                excl = plsc.cumsum(mi) - mi
                pos = jnp.minimum(c + excl, CAP - 1)
                plsc.store_scatter(cord, [pos], v, mask=msk)
                plsc.store_scatter(cidx, [pos], base + iota, mask=msk)
                c = c + plsc.all_reduce_population_count(msk)
            return c
        c = lax.fori_loop(0, NV // 4, gb, zeros16)
        cnt = jnp.minimum(_scal(c), CAP)
        plsc.store_scatter(cord, [jnp.minimum(cnt + iota, CAP - 1)], imaxv,
                           mask=cnt + iota < CAP)
        nv = lax.shift_right_arithmetic(cnt + 15, 4)

        def count_le(t):
            def cb(j2, a):
                return a + plsc.all_reduce_population_count(
                    cord[pl.ds(j2 * 16, 16)] <= t)
            return _scal(lax.fori_loop(0, nv, cb, zeros16))

        neg = count_le(jnp.int32(-1)) >= CTX
        lo0 = jnp.where(neg, INTMIN, jnp.int32(0))
        hi0 = jnp.where(neg, jnp.int32(-1), jnp.int32(IMAX))
        def bs(_, lh):
            lo, hi = lh
            mid = lo + lax.shift_right_arithmetic(hi - lo, 1)
            ok = count_le(mid) >= CTX
            return (jnp.where(ok, lo, mid + 1), jnp.where(ok, mid, hi))
        _, K = lax.fori_loop(0, 32, bs, (lo0, hi0))
        c_lt = jnp.where(K == INTMIN, 0, count_le(K - 1))
        R = CTX - c_lt

        def count_eq_lt(t):
            def cb(j2, a):
                o = cord[pl.ds(j2 * 16, 16)]
                idv = cidx[pl.ds(j2 * 16, 16)]
                return a + plsc.all_reduce_population_count((o == K) & (idv < t))
            return _scal(lax.fori_loop(0, nv, cb, zeros16))

        def bsi(_, lh):
            lo, hi = lh
            mid = lo + lax.shift_right_arithmetic(hi - lo, 1)
            ok = count_eq_lt(mid) >= R
            return (jnp.where(ok, lo, mid + 1), jnp.where(ok, mid, hi))
        _, I = lax.fori_loop(0, 17, bsi, (jnp.int32(0), jnp.int32(NPAD)))

        def ge(j2, c2):
            o = cord[pl.ds(j2 * 16, 16)]
            idv = cidx[pl.ds(j2 * 16, 16)]
            take = (o < K) | ((o == K) & (idv < I))
            mi = take.astype(jnp.int32)
            excl = plsc.cumsum(mi) - mi
            pos = jnp.minimum(c2 + excl, CTX - 1)
            plsc.store_scatter(oidx, [pos], idv, mask=take)
            return c2 + plsc.all_reduce_population_count(take)
        lax.fori_loop(0, nv, ge, zeros16)

        pltpu.async_copy(bank_hbm.at[oidx], ctxbuf, sem).wait()
        pltpu.sync_copy(ctxbuf, ctx_hbm.at[r])
        return carry

    lax.fori_loop(0, RPW, row_body, 0)


def _ctx_body(x_ref, k_ref, ctx_ref, wlab, blab, Wt1, bt1, Wt2,
              plw, plb, Wp1, bp1, Wp2, bp2, gh, bh, Wh, bhd, out_ref):
    ck = ctx_ref[:, :, :D]
    yc = ctx_ref[:, :, D:D + 1]
    kq = k_ref[...]
    sim = 2.0 * jnp.sum(kq[:, None, :] * ck, axis=-1) - jnp.sum(ck * ck, axis=-1)
    m = jnp.max(sim, axis=-1, keepdims=True)
    p = jnp.exp(sim - m)
    p = p / jnp.sum(p, axis=-1, keepdims=True)

    diff = (kq[:, None, :] - ck).reshape(QB * CTX, D)
    t = _mm(jax.nn.relu(_mm(diff, Wt1[...]) + bt1[...]), Wt2[...])
    vals = yc * wlab[...][None] + blab[...][None] + t.reshape(QB, CTX, D)
    ctx_x = jnp.sum(p[:, :, None] * vals, axis=1)

    x2 = x_ref[...] + ctx_x
    X = jnp.broadcast_to(x2[:, None, :], (QB, KE, D))
    mu = jnp.mean(X, axis=-1, keepdims=True)
    var = jnp.mean((X - mu) ** 2, axis=-1, keepdims=True)
    h = (X - mu) / jnp.sqrt(var + EPS) * plw[...][None] + plb[...][None]
    h2 = h.reshape(QB * KE, D)
    h2 = _mm(jax.nn.relu(_mm(h2, Wp1[...]) + bp1[...]), Wp2[...]) + bp2[...]
    X = X + h2.reshape(QB, KE, D)
    o = _ln(X, gh[...][None], bh[...][None])
    o = jax.nn.relu(o).reshape(QB * KE, D)
    out_ref[...] = _mm(o, Wh[...]) + bhd[...]


def _full(shape):
    nd = len(shape)
    return pl.BlockSpec(shape, lambda *_: (0,) * nd)


def kernel(x_num, candidate_x_num, candidate_y, W_lin, b_lin, W_e1, b_e1,
           W_e2, b_e2, g_mix, b_mix, W_K, b_K, w_label, b_label, W_t1, b_t1,
           W_t2, pln_w, pln_b, W_p1, b_p1, W_p2, b_p2, g_hln, b_hln, W_head,
           b_head, is_train):
    f32 = jnp.float32
    cxp = jnp.zeros((NPAD, DIN), f32).at[:N].set(candidate_x_num)
    y128 = jnp.broadcast_to(
        jnp.zeros((NPAD,), f32).at[:N].set(candidate_y)[:, None], (NPAD, DIN))
    bl = b_lin.reshape(1, D)
    b1 = b_e1.reshape(1, DB)
    b2 = b_e2.reshape(1, D)
    g = g_mix.reshape(1, D)
    bm = b_mix.reshape(1, D)
    bK = b_K.reshape(1, D)

    wspecs = [_full(W_lin.shape), _full((1, D)), _full(W_e1.shape),
              _full((1, DB)), _full(W_e2.shape), _full((1, D)),
              _full((1, D)), _full((1, D)), _full(W_K.shape), _full((1, D))]
    wargs = (W_lin, bl, W_e1, b1, W_e2, b2, g, bm, W_K, bK)

    bank = pl.pallas_call(
        _bank_body,
        grid=(NBLK,),
        in_specs=[pl.BlockSpec((RB, DIN), lambda j: (j, 0)),
                  pl.BlockSpec((RB, DIN), lambda j: (j, 0))] + wspecs,
        out_specs=pl.BlockSpec((RB, BANKW), lambda j: (j, 0)),
        out_shape=jax.ShapeDtypeStruct((NPAD, BANKW), f32),
    )(cxp, y128, *wargs)

    xq, kq = pl.pallas_call(
        _query_body,
        in_specs=[_full((B, DIN))] + wspecs,
        out_specs=[_full((B, D)), _full((B, D))],
        out_shape=[jax.ShapeDtypeStruct((B, D), f32),
                   jax.ShapeDtypeStruct((B, D), f32)],
    )(x_num, *wargs)

    ordm = pl.pallas_call(
        _dist_body,
        grid=(NBLK,),
        in_specs=[_full((B, D)), pl.BlockSpec((RB, BANKW), lambda j: (j, 0))],
        out_specs=pl.BlockSpec((B, RB), lambda j: (0, j)),
        out_shape=jax.ShapeDtypeStruct((B, NPAD), jnp.int32),
    )(kq, bank)

    ctx = pl.kernel(
        _sc_body,
        out_type=jax.ShapeDtypeStruct((B, CTX, BANKW), f32),
        mesh=plsc.VectorSubcoreMesh(core_axis_name="c", subcore_axis_name="s"),
        compiler_params=pltpu.CompilerParams(needs_layout_passes=False),
        scratch_types=[
            pltpu.VMEM((NPAD,), jnp.int32),
            pltpu.VMEM((CAP,), jnp.int32),
            pltpu.VMEM((CAP,), jnp.int32),
            pltpu.VMEM((CTX,), jnp.int32),
            pltpu.VMEM((CTX, BANKW), f32),
            pltpu.SemaphoreType.DMA,
        ],
    )(ordm, bank)

    out = pl.pallas_call(
        _ctx_body,
        grid=(B // QB,),
        in_specs=[pl.BlockSpec((QB, D), lambda j: (j, 0)),
                  pl.BlockSpec((QB, D), lambda j: (j, 0)),
                  pl.BlockSpec((QB, CTX, BANKW), lambda j: (j, 0, 0)),
                  _full((1, D)), _full((1, D)), _full(W_t1.shape),
                  _full((1, DB)), _full(W_t2.shape), _full(pln_w.shape),
                  _full(pln_b.shape), _full(W_p1.shape), _full((1, DB)),
                  _full(W_p2.shape), _full((1, D)), _full((1, D)),
                  _full((1, D)), _full(W_head.shape), _full((1, 1))],
        out_specs=pl.BlockSpec((QB * KE, 1), lambda j: (j, 0)),
        out_shape=jax.ShapeDtypeStruct((B * KE, 1), f32),
    )(xq, kq, ctx, w_label.reshape(1, D), b_label.reshape(1, D), W_t1,
      b_t1.reshape(1, DB), W_t2, pln_w, pln_b, W_p1, b_p1.reshape(1, DB),
      W_p2, b_p2.reshape(1, D), g_hln.reshape(1, D), b_hln.reshape(1, D),
      W_head, b_head.reshape(1, 1))

    return out.reshape(B, KE, 1)


# speedup vs baseline: 9.5353x; 1.8997x over previous
import functools

import jax
import jax.numpy as jnp
from jax import lax
from jax.experimental import pallas as pl
from jax.experimental.pallas import tpu as pltpu
from jax.experimental.pallas import tpu_sc as plsc

B = 1024
N = 50000
NPAD = 50176
DIN = 128
D = 256
DB = 512
CTX = 96
KE = 4
EPS = 1e-5
RB = 1024
NBLK = NPAD // RB
QB = 16
BANKW = D + DIN
IMAX = 2147483647
PADORD = 0x7F000000

NW = 32
RPW = B // NW
NV = NPAD // 16
CAP = 8192
HIGHEST = lax.Precision.DEFAULT


def _ln(x, g, b):
    mu = jnp.mean(x, axis=-1, keepdims=True)
    var = jnp.mean((x - mu) ** 2, axis=-1, keepdims=True)
    return (x - mu) / jnp.sqrt(var + EPS) * g + b


def _mm(a, b):
    return lax.dot_general(a, b, (((1,), (0,)), ((), ())),
                           preferred_element_type=jnp.float32,
                           precision=HIGHEST)


def _encode(cx, Wl, bl, W1, b1, W2, b2, g, bm, WK, bK):
    x = _mm(cx, Wl) + bl
    h = _mm(jax.nn.relu(_mm(x, W1) + b1), W2) + b2
    x = x + h
    k = _mm(_ln(x, g, bm), WK) + bK
    return x, k


def _bank_body(cx_ref, y_ref, Wl, bl, W1, b1, W2, b2, g, bm, WK, bK, out_ref):
    _, ck = _encode(cx_ref[...], Wl[...], bl[...], W1[...], b1[...],
                    W2[...], b2[...], g[...], bm[...], WK[...], bK[...])
    out_ref[:, :D] = ck
    out_ref[:, D:] = y_ref[...]


def _query_body(x_ref, Wl, bl, W1, b1, W2, b2, g, bm, WK, bK, x_out, k_out):
    x, k = _encode(x_ref[...], Wl[...], bl[...], W1[...], b1[...],
                   W2[...], b2[...], g[...], bm[...], WK[...], bK[...])
    x_out[...] = x
    k_out[...] = k


def _dist_body(k_ref, bank_ref, o_ref):
    j = pl.program_id(0)
    ck = bank_ref[:, :D]
    kq = k_ref[...]
    dot = lax.dot_general(kq, ck, (((1,), (1,)), ((), ())),
                          preferred_element_type=jnp.float32,
                          precision=HIGHEST)
    cn = lax.dot_general(jnp.ones((1, D), jnp.float32), ck * ck,
                         (((1,), (1,)), ((), ())),
                         preferred_element_type=jnp.float32,
                         precision=lax.Precision.HIGHEST)
    kn = jnp.sum(kq * kq, axis=1, keepdims=True)
    s = (kn - 2.0 * dot) + cn
    bits = lax.bitcast_convert_type(s, jnp.int32)
    o = jnp.where(bits < 0, bits ^ 0x7FFFFFFF, bits)
    col = j * RB + lax.broadcasted_iota(jnp.int32, (1, RB), 1)
    o_ref[...] = jnp.where(col >= N, PADORD, o)


def _sc_body(ord_hbm, bank_hbm, ctx_hbm, srow, cord, cidx, oidx, ctxbuf, sem,
             sem2):
    wid = lax.axis_index("s") * 2 + lax.axis_index("c")
    iota = lax.iota(jnp.int32, 16)
    zeros16 = jnp.zeros((16,), jnp.int32)
    imaxv = jnp.full((16,), IMAX, jnp.int32)
    INTMIN = jnp.int32(-IMAX - 1)

    def _scal(v):
        return v[0]

    def row_body(rr, carry):
        r = wid * RPW + rr
        pltpu.sync_copy(ord_hbm.at[r], srow)

        def ga(g, accs):
            base = g * 96
            return tuple(jnp.minimum(accs[j], srow[pl.ds(base + j * 16, 16)])
                         for j in range(6))
        accs = lax.fori_loop(0, 522, ga, (imaxv,) * 6)
        for j in range(4):
            v = srow[pl.ds(522 * 96 + j * 16, 16)]
            accs = tuple(jnp.minimum(accs[i], v) if i == j else accs[i]
                         for i in range(6))
        m = accs[0]
        mn = accs[0]
        for j in range(1, 6):
            m = jnp.maximum(m, accs[j])
            mn = jnp.minimum(mn, accs[j])
        U = m[0]
        LO = mn[0]
        for l in range(1, 16):
            U = jnp.maximum(U, m[l])
            LO = jnp.minimum(LO, mn[l])

        def gb(i, c):
            base = i * 64
            vs = [srow[pl.ds(base + u * 16, 16)] for u in range(4)]
            msks = [v <= U for v in vs]
            incls = [plsc.cumsum(jnp.where(mk, 1, 0)) for mk in msks]
            cm1 = jnp.minimum(c, CAP - 65) - 1
            for u in range(4):
                pos = cm1 + incls[u]
                plsc.store_scatter(cord, [pos], vs[u], mask=msks[u])
                plsc.store_scatter(cidx, [pos], base + u * 16 + iota,
                                   mask=msks[u])
                cm1 = cm1 + plsc.all_reduce_population_count(msks[u])
            return cm1 + 1
        c = lax.fori_loop(0, NV // 4, gb, zeros16)
        cnt = jnp.minimum(_scal(c), CAP)
        plsc.store_scatter(cord, [jnp.minimum(cnt + iota, CAP - 1)], imaxv,
                           mask=cnt + iota < CAP)
        plsc.store_scatter(cord, [jnp.minimum(cnt + 16 + iota, CAP - 1)],
                           imaxv, mask=cnt + 16 + iota < CAP)
        nv2 = lax.shift_right_arithmetic(cnt + 31, 5)

        def count_le(t):
            def cb(j2, a):
                a = a + plsc.all_reduce_population_count(
                    cord[pl.ds(j2 * 32, 16)] <= t)
                return a + plsc.all_reduce_population_count(
                    cord[pl.ds(j2 * 32 + 16, 16)] <= t)
            return _scal(lax.fori_loop(0, nv2, cb, zeros16))

        def bs(lh):
            lo, hi = lh
            mid = lo + lax.shift_right_arithmetic(hi - lo, 1)
            ok = count_le(mid) >= CTX
            return (jnp.where(ok, lo, mid + 1), jnp.where(ok, mid, hi))
        _, K = lax.while_loop(lambda lh: lh[0] < lh[1], bs, (LO, U))
        c_lt = jnp.where(K == INTMIN, 0, count_le(K - 1))
        R = CTX - c_lt

        def count_eq_lt(t):
            def cb(j2, a):
                o = cord[pl.ds(j2 * 32, 16)]
                o2 = cord[pl.ds(j2 * 32 + 16, 16)]
                idv = cidx[pl.ds(j2 * 32, 16)]
                idv2 = cidx[pl.ds(j2 * 32 + 16, 16)]
                a = a + plsc.all_reduce_population_count((o == K) & (idv < t))
                return a + plsc.all_reduce_population_count(
                    (o2 == K) & (idv2 < t))
            return _scal(lax.fori_loop(0, nv2, cb, zeros16))

        def bsi(lh):
            lo, hi = lh
            mid = lo + lax.shift_right_arithmetic(hi - lo, 1)
            ok = count_eq_lt(mid) >= R
            return (jnp.where(ok, lo, mid + 1), jnp.where(ok, mid, hi))
        _, I = lax.while_loop(lambda lh: lh[0] < lh[1], bsi,
                              (jnp.int32(0), jnp.int32(NPAD)))
        nv = lax.shift_right_arithmetic(cnt + 15, 4)

        def ge(j2, c2):
            o = cord[pl.ds(j2 * 16, 16)]
            idv = cidx[pl.ds(j2 * 16, 16)]
            take = (o < K) | ((o == K) & (idv < I))
            mi = take.astype(jnp.int32)
            excl = plsc.cumsum(mi) - mi
            pos = jnp.minimum(c2 + excl, CTX - 1)
            plsc.store_scatter(oidx, [pos], idv, mask=take)
            return c2 + plsc.all_reduce_population_count(take)
        lax.fori_loop(0, nv, ge, zeros16)

        @pl.when(rr > 0)
        def _drain_prev():
            pltpu.make_async_copy(ctxbuf, ctx_hbm.at[r - 1], sem2).wait()

        pltpu.async_copy(bank_hbm.at[oidx], ctxbuf, sem).wait()
        pltpu.async_copy(ctxbuf, ctx_hbm.at[r], sem2)
        return carry

    lax.fori_loop(0, RPW, row_body, 0)
    pltpu.make_async_copy(ctxbuf, ctx_hbm.at[wid * RPW + RPW - 1], sem2).wait()


def _ctx_body(x_ref, k_ref, ctx_ref, wlab, blab, Wt1, bt1, Wt2,
              plw, plb, Wp1, bp1, Wp2, bp2, gh, bh, Wh, bhd, out_ref):
    ck = ctx_ref[:, :, :D]
    yc = ctx_ref[:, :, D:D + 1]
    kq = k_ref[...]
    sim = 2.0 * jnp.sum(kq[:, None, :] * ck, axis=-1) - jnp.sum(ck * ck, axis=-1)
    m = jnp.max(sim, axis=-1, keepdims=True)
    p = jnp.exp(sim - m)
    p = p / jnp.sum(p, axis=-1, keepdims=True)

    diff = (kq[:, None, :] - ck).reshape(QB * CTX, D)
    t = _mm(jax.nn.relu(_mm(diff, Wt1[...]) + bt1[...]), Wt2[...])
    vals = yc * wlab[...][None] + blab[...][None] + t.reshape(QB, CTX, D)
    ctx_x = jnp.sum(p[:, :, None] * vals, axis=1)

    x2 = x_ref[...] + ctx_x
    X = jnp.broadcast_to(x2[:, None, :], (QB, KE, D))
    mu = jnp.mean(X, axis=-1, keepdims=True)
    var = jnp.mean((X - mu) ** 2, axis=-1, keepdims=True)
    h = (X - mu) / jnp.sqrt(var + EPS) * plw[...][None] + plb[...][None]
    h2 = h.reshape(QB * KE, D)
    h2 = _mm(jax.nn.relu(_mm(h2, Wp1[...]) + bp1[...]), Wp2[...]) + bp2[...]
    X = X + h2.reshape(QB, KE, D)
    o = _ln(X, gh[...][None], bh[...][None])
    o = jax.nn.relu(o).reshape(QB * KE, D)
    out_ref[...] = _mm(o, Wh[...]) + bhd[...]


def _full(shape):
    nd = len(shape)
    return pl.BlockSpec(shape, lambda *_: (0,) * nd)


def kernel(x_num, candidate_x_num, candidate_y, W_lin, b_lin, W_e1, b_e1,
           W_e2, b_e2, g_mix, b_mix, W_K, b_K, w_label, b_label, W_t1, b_t1,
           W_t2, pln_w, pln_b, W_p1, b_p1, W_p2, b_p2, g_hln, b_hln, W_head,
           b_head, is_train):
    f32 = jnp.float32
    cxp = jnp.zeros((NPAD, DIN), f32).at[:N].set(candidate_x_num)
    y128 = jnp.broadcast_to(
        jnp.zeros((NPAD,), f32).at[:N].set(candidate_y)[:, None], (NPAD, DIN))
    bl = b_lin.reshape(1, D)
    b1 = b_e1.reshape(1, DB)
    b2 = b_e2.reshape(1, D)
    g = g_mix.reshape(1, D)
    bm = b_mix.reshape(1, D)
    bK = b_K.reshape(1, D)

    wspecs = [_full(W_lin.shape), _full((1, D)), _full(W_e1.shape),
              _full((1, DB)), _full(W_e2.shape), _full((1, D)),
              _full((1, D)), _full((1, D)), _full(W_K.shape), _full((1, D))]
    wargs = (W_lin, bl, W_e1, b1, W_e2, b2, g, bm, W_K, bK)

    bank = pl.pallas_call(
        _bank_body,
        grid=(NBLK,),
        in_specs=[pl.BlockSpec((RB, DIN), lambda j: (j, 0)),
                  pl.BlockSpec((RB, DIN), lambda j: (j, 0))] + wspecs,
        out_specs=pl.BlockSpec((RB, BANKW), lambda j: (j, 0)),
        out_shape=jax.ShapeDtypeStruct((NPAD, BANKW), f32),
    )(cxp, y128, *wargs)

    xq, kq = pl.pallas_call(
        _query_body,
        in_specs=[_full((B, DIN))] + wspecs,
        out_specs=[_full((B, D)), _full((B, D))],
        out_shape=[jax.ShapeDtypeStruct((B, D), f32),
                   jax.ShapeDtypeStruct((B, D), f32)],
    )(x_num, *wargs)

    ordm = pl.pallas_call(
        _dist_body,
        grid=(NBLK,),
        in_specs=[_full((B, D)), pl.BlockSpec((RB, BANKW), lambda j: (j, 0))],
        out_specs=pl.BlockSpec((B, RB), lambda j: (0, j)),
        out_shape=jax.ShapeDtypeStruct((B, NPAD), jnp.int32),
    )(kq, bank)

    ctx = pl.kernel(
        _sc_body,
        out_type=jax.ShapeDtypeStruct((B, CTX, BANKW), f32),
        mesh=plsc.VectorSubcoreMesh(core_axis_name="c", subcore_axis_name="s"),
        compiler_params=pltpu.CompilerParams(needs_layout_passes=False),
        scratch_types=[
            pltpu.VMEM((NPAD,), jnp.int32),
            pltpu.VMEM((CAP,), jnp.int32),
            pltpu.VMEM((CAP,), jnp.int32),
            pltpu.VMEM((CTX,), jnp.int32),
            pltpu.VMEM((CTX, BANKW), f32),
            pltpu.SemaphoreType.DMA,
            pltpu.SemaphoreType.DMA,
        ],
    )(ordm, bank)

    out = pl.pallas_call(
        _ctx_body,
        grid=(B // QB,),
        in_specs=[pl.BlockSpec((QB, D), lambda j: (j, 0)),
                  pl.BlockSpec((QB, D), lambda j: (j, 0)),
                  pl.BlockSpec((QB, CTX, BANKW), lambda j: (j, 0, 0)),
                  _full((1, D)), _full((1, D)), _full(W_t1.shape),
                  _full((1, DB)), _full(W_t2.shape), _full(pln_w.shape),
                  _full(pln_b.shape), _full(W_p1.shape), _full((1, DB)),
                  _full(W_p2.shape), _full((1, D)), _full((1, D)),
                  _full((1, D)), _full(W_head.shape), _full((1, 1))],
        out_specs=pl.BlockSpec((QB * KE, 1), lambda j: (j, 0)),
        out_shape=jax.ShapeDtypeStruct((B * KE, 1), f32),
    )(xq, kq, ctx, w_label.reshape(1, D), b_label.reshape(1, D), W_t1,
      b_t1.reshape(1, DB), W_t2, pln_w, pln_b, W_p1, b_p1.reshape(1, DB),
      W_p2, b_p2.reshape(1, D), g_hln.reshape(1, D), b_hln.reshape(1, D),
      W_head, b_head.reshape(1, 1))

    return out.reshape(B, KE, 1)


# speedup vs baseline: 10.2560x; 1.0756x over previous
import functools

import jax
import jax.numpy as jnp
from jax import lax
from jax.experimental import pallas as pl
from jax.experimental.pallas import tpu as pltpu
from jax.experimental.pallas import tpu_sc as plsc

B = 1024
N = 50000
NPAD = 50176
DIN = 128
D = 256
DB = 512
CTX = 96
KE = 4
EPS = 1e-5
RB = 1024
NBLK = NPAD // RB
QB = 16
BANKW = D + DIN
IMAX = 2147483647
PADORD = 0x7F000000

NW = 32
RPW = B // NW
NV = NPAD // 16
CAP = 8192
HIGHEST = lax.Precision.DEFAULT


def _ln(x, g, b):
    mu = jnp.mean(x, axis=-1, keepdims=True)
    var = jnp.mean((x - mu) ** 2, axis=-1, keepdims=True)
    return (x - mu) / jnp.sqrt(var + EPS) * g + b


def _mm(a, b):
    return lax.dot_general(a, b, (((1,), (0,)), ((), ())),
                           preferred_element_type=jnp.float32,
                           precision=HIGHEST)


def _encode(cx, Wl, bl, W1, b1, W2, b2, g, bm, WK, bK):
    x = _mm(cx, Wl) + bl
    h = _mm(jax.nn.relu(_mm(x, W1) + b1), W2) + b2
    x = x + h
    k = _mm(_ln(x, g, bm), WK) + bK
    return x, k


def _bank_body(cx_ref, y_ref, Wl, bl, W1, b1, W2, b2, g, bm, WK, bK, out_ref):
    _, ck = _encode(cx_ref[...], Wl[...], bl[...], W1[...], b1[...],
                    W2[...], b2[...], g[...], bm[...], WK[...], bK[...])
    out_ref[:, :D] = ck
    out_ref[:, D:] = y_ref[...]


def _query_body(x_ref, Wl, bl, W1, b1, W2, b2, g, bm, WK, bK, x_out, k_out):
    x, k = _encode(x_ref[...], Wl[...], bl[...], W1[...], b1[...],
                   W2[...], b2[...], g[...], bm[...], WK[...], bK[...])
    x_out[...] = x
    k_out[...] = k


def _dist_body(k_ref, bank_ref, o_ref):
    j = pl.program_id(0)
    ck = bank_ref[:, :D]
    kq = k_ref[...]
    dot = lax.dot_general(kq, ck, (((1,), (1,)), ((), ())),
                          preferred_element_type=jnp.float32,
                          precision=HIGHEST)
    cn = lax.dot_general(jnp.ones((1, D), jnp.float32), ck * ck,
                         (((1,), (1,)), ((), ())),
                         preferred_element_type=jnp.float32,
                         precision=lax.Precision.HIGHEST)
    kn = jnp.sum(kq * kq, axis=1, keepdims=True)
    s = (kn - 2.0 * dot) + cn
    bits = lax.bitcast_convert_type(s, jnp.int32)
    o = jnp.where(bits < 0, bits ^ 0x7FFFFFFF, bits)
    col = j * RB + lax.broadcasted_iota(jnp.int32, (1, RB), 1)
    o_ref[...] = jnp.where(col >= N, PADORD, o)


def _sc_body(ord_hbm, bank_hbm, ctx_hbm, srow, cord, cidx, oidx, ctxbuf, sem,
             sem2, sem3):
    wid = lax.axis_index("s") * 2 + lax.axis_index("c")
    iota = lax.iota(jnp.int32, 16)
    zeros16 = jnp.zeros((16,), jnp.int32)
    imaxv = jnp.full((16,), IMAX, jnp.int32)
    INTMIN = jnp.int32(-IMAX - 1)

    def _scal(v):
        return v[0]

    def row_body(rr, carry):
        r = wid * RPW + rr
        pltpu.make_async_copy(ord_hbm.at[r], srow, sem3).wait()

        def ga(g, accs):
            base = g * 96
            return tuple(jnp.minimum(accs[j], srow[pl.ds(base + j * 16, 16)])
                         for j in range(6))
        accs = lax.fori_loop(0, 522, ga, (imaxv,) * 6)
        for j in range(4):
            v = srow[pl.ds(522 * 96 + j * 16, 16)]
            accs = tuple(jnp.minimum(accs[i], v) if i == j else accs[i]
                         for i in range(6))
        m = accs[0]
        mn = accs[0]
        for j in range(1, 6):
            m = jnp.maximum(m, accs[j])
            mn = jnp.minimum(mn, accs[j])
        U = m[0]
        LO = mn[0]
        for l in range(1, 16):
            U = jnp.maximum(U, m[l])
            LO = jnp.minimum(LO, mn[l])

        def gb(i, c):
            base = i * 64
            vs = [srow[pl.ds(base + u * 16, 16)] for u in range(4)]
            msks = [v <= U for v in vs]
            incls = [plsc.cumsum(jnp.where(mk, 1, 0)) for mk in msks]
            cm1 = jnp.minimum(c, CAP - 65) - 1
            for u in range(4):
                pos = cm1 + incls[u]
                plsc.store_scatter(cord, [pos], vs[u], mask=msks[u])
                plsc.store_scatter(cidx, [pos], base + u * 16 + iota,
                                   mask=msks[u])
                cm1 = cm1 + plsc.all_reduce_population_count(msks[u])
            return cm1 + 1
        c = lax.fori_loop(0, NV // 4, gb, zeros16)

        @pl.when(rr < RPW - 1)
        def _prefetch_next():
            pltpu.async_copy(ord_hbm.at[r + 1], srow, sem3)

        cnt = jnp.minimum(_scal(c), CAP)
        plsc.store_scatter(cord, [jnp.minimum(cnt + iota, CAP - 1)], imaxv,
                           mask=cnt + iota < CAP)
        plsc.store_scatter(cord, [jnp.minimum(cnt + 16 + iota, CAP - 1)],
                           imaxv, mask=cnt + 16 + iota < CAP)
        nv2 = lax.shift_right_arithmetic(cnt + 31, 5)

        def count_le(t):
            def cb(j2, a):
                a = a + plsc.all_reduce_population_count(
                    cord[pl.ds(j2 * 32, 16)] <= t)
                return a + plsc.all_reduce_population_count(
                    cord[pl.ds(j2 * 32 + 16, 16)] <= t)
            return _scal(lax.fori_loop(0, nv2, cb, zeros16))

        def bs(lh):
            lo, hi = lh
            mid = lo + lax.shift_right_arithmetic(hi - lo, 1)
            ok = count_le(mid) >= CTX
            return (jnp.where(ok, lo, mid + 1), jnp.where(ok, mid, hi))
        _, K = lax.while_loop(lambda lh: lh[0] < lh[1], bs, (LO, U))
        c_lt = jnp.where(K == INTMIN, 0, count_le(K - 1))
        R = CTX - c_lt

        def count_eq_lt(t):
            def cb(j2, a):
                o = cord[pl.ds(j2 * 32, 16)]
                o2 = cord[pl.ds(j2 * 32 + 16, 16)]
                idv = cidx[pl.ds(j2 * 32, 16)]
                idv2 = cidx[pl.ds(j2 * 32 + 16, 16)]
                a = a + plsc.all_reduce_population_count((o == K) & (idv < t))
                return a + plsc.all_reduce_population_count(
                    (o2 == K) & (idv2 < t))
            return _scal(lax.fori_loop(0, nv2, cb, zeros16))

        def bsi(lh):
            lo, hi = lh
            mid = lo + lax.shift_right_arithmetic(hi - lo, 1)
            ok = count_eq_lt(mid) >= R
            return (jnp.where(ok, lo, mid + 1), jnp.where(ok, mid, hi))
        _, I = lax.while_loop(lambda lh: lh[0] < lh[1], bsi,
                              (jnp.int32(0), jnp.int32(NPAD)))
        nv = lax.shift_right_arithmetic(cnt + 15, 4)

        def ge(j2, c2):
            o = cord[pl.ds(j2 * 16, 16)]
            idv = cidx[pl.ds(j2 * 16, 16)]
            take = (o < K) | ((o == K) & (idv < I))
            mi = take.astype(jnp.int32)
            excl = plsc.cumsum(mi) - mi
            pos = jnp.minimum(c2 + excl, CTX - 1)
            plsc.store_scatter(oidx, [pos], idv, mask=take)
            return c2 + plsc.all_reduce_population_count(take)
        lax.fori_loop(0, nv, ge, zeros16)

        @pl.when(rr > 0)
        def _drain_prev():
            pltpu.make_async_copy(ctxbuf, ctx_hbm.at[r - 1], sem2).wait()

        pltpu.async_copy(bank_hbm.at[oidx], ctxbuf, sem).wait()
        pltpu.async_copy(ctxbuf, ctx_hbm.at[r], sem2)
        return carry

    pltpu.async_copy(ord_hbm.at[wid * RPW], srow, sem3)
    lax.fori_loop(0, RPW, row_body, 0)
    pltpu.make_async_copy(ctxbuf, ctx_hbm.at[wid * RPW + RPW - 1], sem2).wait()


def _ctx_body(x_ref, k_ref, ctx_ref, wlab, blab, Wt1, bt1, Wt2,
              plw, plb, Wp1, bp1, Wp2, bp2, gh, bh, Wh, bhd, out_ref):
    ck = ctx_ref[:, :, :D]
    yc = ctx_ref[:, :, D:D + 1]
    kq = k_ref[...]
    sim = 2.0 * jnp.sum(kq[:, None, :] * ck, axis=-1) - jnp.sum(ck * ck, axis=-1)
    m = jnp.max(sim, axis=-1, keepdims=True)
    p = jnp.exp(sim - m)
    p = p / jnp.sum(p, axis=-1, keepdims=True)

    diff = (kq[:, None, :] - ck).reshape(QB * CTX, D)
    t = _mm(jax.nn.relu(_mm(diff, Wt1[...]) + bt1[...]), Wt2[...])
    vals = yc * wlab[...][None] + blab[...][None] + t.reshape(QB, CTX, D)
    ctx_x = jnp.sum(p[:, :, None] * vals, axis=1)

    x2 = x_ref[...] + ctx_x
    X = jnp.broadcast_to(x2[:, None, :], (QB, KE, D))
    mu = jnp.mean(X, axis=-1, keepdims=True)
    var = jnp.mean((X - mu) ** 2, axis=-1, keepdims=True)
    h = (X - mu) / jnp.sqrt(var + EPS) * plw[...][None] + plb[...][None]
    h2 = h.reshape(QB * KE, D)
    h2 = _mm(jax.nn.relu(_mm(h2, Wp1[...]) + bp1[...]), Wp2[...]) + bp2[...]
    X = X + h2.reshape(QB, KE, D)
    o = _ln(X, gh[...][None], bh[...][None])
    o = jax.nn.relu(o).reshape(QB * KE, D)
    out_ref[...] = _mm(o, Wh[...]) + bhd[...]


def _full(shape):
    nd = len(shape)
    return pl.BlockSpec(shape, lambda *_: (0,) * nd)


def kernel(x_num, candidate_x_num, candidate_y, W_lin, b_lin, W_e1, b_e1,
           W_e2, b_e2, g_mix, b_mix, W_K, b_K, w_label, b_label, W_t1, b_t1,
           W_t2, pln_w, pln_b, W_p1, b_p1, W_p2, b_p2, g_hln, b_hln, W_head,
           b_head, is_train):
    f32 = jnp.float32
    cxp = jnp.zeros((NPAD, DIN), f32).at[:N].set(candidate_x_num)
    y128 = jnp.broadcast_to(
        jnp.zeros((NPAD,), f32).at[:N].set(candidate_y)[:, None], (NPAD, DIN))
    bl = b_lin.reshape(1, D)
    b1 = b_e1.reshape(1, DB)
    b2 = b_e2.reshape(1, D)
    g = g_mix.reshape(1, D)
    bm = b_mix.reshape(1, D)
    bK = b_K.reshape(1, D)

    wspecs = [_full(W_lin.shape), _full((1, D)), _full(W_e1.shape),
              _full((1, DB)), _full(W_e2.shape), _full((1, D)),
              _full((1, D)), _full((1, D)), _full(W_K.shape), _full((1, D))]
    wargs = (W_lin, bl, W_e1, b1, W_e2, b2, g, bm, W_K, bK)

    bank = pl.pallas_call(
        _bank_body,
        grid=(NBLK,),
        in_specs=[pl.BlockSpec((RB, DIN), lambda j: (j, 0)),
                  pl.BlockSpec((RB, DIN), lambda j: (j, 0))] + wspecs,
        out_specs=pl.BlockSpec((RB, BANKW), lambda j: (j, 0)),
        out_shape=jax.ShapeDtypeStruct((NPAD, BANKW), f32),
    )(cxp, y128, *wargs)

    xq, kq = pl.pallas_call(
        _query_body,
        in_specs=[_full((B, DIN))] + wspecs,
        out_specs=[_full((B, D)), _full((B, D))],
        out_shape=[jax.ShapeDtypeStruct((B, D), f32),
                   jax.ShapeDtypeStruct((B, D), f32)],
    )(x_num, *wargs)

    ordm = pl.pallas_call(
        _dist_body,
        grid=(NBLK,),
        in_specs=[_full((B, D)), pl.BlockSpec((RB, BANKW), lambda j: (j, 0))],
        out_specs=pl.BlockSpec((B, RB), lambda j: (0, j)),
        out_shape=jax.ShapeDtypeStruct((B, NPAD), jnp.int32),
    )(kq, bank)

    ctx = pl.kernel(
        _sc_body,
        out_type=jax.ShapeDtypeStruct((B, CTX, BANKW), f32),
        mesh=plsc.VectorSubcoreMesh(core_axis_name="c", subcore_axis_name="s"),
        compiler_params=pltpu.CompilerParams(needs_layout_passes=False),
        scratch_types=[
            pltpu.VMEM((NPAD,), jnp.int32),
            pltpu.VMEM((CAP,), jnp.int32),
            pltpu.VMEM((CAP,), jnp.int32),
            pltpu.VMEM((CTX,), jnp.int32),
            pltpu.VMEM((CTX, BANKW), f32),
            pltpu.SemaphoreType.DMA,
            pltpu.SemaphoreType.DMA,
            pltpu.SemaphoreType.DMA,
        ],
    )(ordm, bank)

    out = pl.pallas_call(
        _ctx_body,
        grid=(B // QB,),
        in_specs=[pl.BlockSpec((QB, D), lambda j: (j, 0)),
                  pl.BlockSpec((QB, D), lambda j: (j, 0)),
                  pl.BlockSpec((QB, CTX, BANKW), lambda j: (j, 0, 0)),
                  _full((1, D)), _full((1, D)), _full(W_t1.shape),
                  _full((1, DB)), _full(W_t2.shape), _full(pln_w.shape),
                  _full(pln_b.shape), _full(W_p1.shape), _full((1, DB)),
                  _full(W_p2.shape), _full((1, D)), _full((1, D)),
                  _full((1, D)), _full(W_head.shape), _full((1, 1))],
        out_specs=pl.BlockSpec((QB * KE, 1), lambda j: (j, 0)),
        out_shape=jax.ShapeDtypeStruct((B * KE, 1), f32),
    )(xq, kq, ctx, w_label.reshape(1, D), b_label.reshape(1, D), W_t1,
      b_t1.reshape(1, DB), W_t2, pln_w, pln_b, W_p1, b_p1.reshape(1, DB),
      W_p2, b_p2.reshape(1, D), g_hln.reshape(1, D), b_hln.reshape(1, D),
      W_head, b_head.reshape(1, 1))

    return out.reshape(B, KE, 1)


# speedup vs baseline: 11.4331x; 1.1148x over previous
import functools

import jax
import jax.numpy as jnp
from jax import lax
from jax.experimental import pallas as pl
from jax.experimental.pallas import tpu as pltpu
from jax.experimental.pallas import tpu_sc as plsc

B = 1024
N = 50000
NPAD = 50176
DIN = 128
D = 256
DB = 512
CTX = 96
KE = 4
EPS = 1e-5
RB = 1024
NBLK = NPAD // RB
QB = 16
BANKW = D + DIN
IMAX = 2147483647
PADORD = 0x7F000000

NW = 32
RPW = B // NW
NV = NPAD // 16
CAP = 8192
HIGHEST = lax.Precision.DEFAULT


def _ln(x, g, b):
    mu = jnp.mean(x, axis=-1, keepdims=True)
    var = jnp.mean((x - mu) ** 2, axis=-1, keepdims=True)
    return (x - mu) / jnp.sqrt(var + EPS) * g + b


def _mm(a, b):
    return lax.dot_general(a, b, (((1,), (0,)), ((), ())),
                           preferred_element_type=jnp.float32,
                           precision=HIGHEST)


def _encode(cx, Wl, bl, W1, b1, W2, b2, g, bm, WK, bK):
    x = _mm(cx, Wl) + bl
    h = _mm(jax.nn.relu(_mm(x, W1) + b1), W2) + b2
    x = x + h
    k = _mm(_ln(x, g, bm), WK) + bK
    return x, k


def _bank_body(cx_ref, y_ref, Wl, bl, W1, b1, W2, b2, g, bm, WK, bK, out_ref):
    _, ck = _encode(cx_ref[...], Wl[...], bl[...], W1[...], b1[...],
                    W2[...], b2[...], g[...], bm[...], WK[...], bK[...])
    out_ref[:, :D] = ck
    out_ref[:, D:] = y_ref[...]


def _query_body(x_ref, Wl, bl, W1, b1, W2, b2, g, bm, WK, bK, x_out, k_out):
    x, k = _encode(x_ref[...], Wl[...], bl[...], W1[...], b1[...],
                   W2[...], b2[...], g[...], bm[...], WK[...], bK[...])
    x_out[...] = x
    k_out[...] = k


def _dist_body(k_ref, bank_ref, o_ref):
    j = pl.program_id(0)
    ck = bank_ref[:, :D]
    kq = k_ref[...]
    dot = lax.dot_general(kq, ck, (((1,), (1,)), ((), ())),
                          preferred_element_type=jnp.float32,
                          precision=HIGHEST)
    cn = lax.dot_general(jnp.ones((1, D), jnp.float32), ck * ck,
                         (((1,), (1,)), ((), ())),
                         preferred_element_type=jnp.float32,
                         precision=lax.Precision.HIGHEST)
    kn = jnp.sum(kq * kq, axis=1, keepdims=True)
    s = (kn - 2.0 * dot) + cn
    bits = lax.bitcast_convert_type(s, jnp.int32)
    o = jnp.where(bits < 0, bits ^ 0x7FFFFFFF, bits)
    col = j * RB + lax.broadcasted_iota(jnp.int32, (1, RB), 1)
    o_ref[...] = jnp.where(col >= N, PADORD, o)


def _make_sc_body(rpw):
    def _sc_body(ord_hbm, bank_hbm, ctx_hbm, srow, cord, cidx, oidx, ctxbuf,
                 sem, sem2, sem3):
        return _sc_rows(rpw, ord_hbm, bank_hbm, ctx_hbm, srow, cord, cidx,
                        oidx, ctxbuf, sem, sem2, sem3)
    return _sc_body


def _sc_rows(RPW, ord_hbm, bank_hbm, ctx_hbm, srow, cord, cidx, oidx, ctxbuf,
             sem, sem2, sem3):
    wid = lax.axis_index("s") * 2 + lax.axis_index("c")
    iota = lax.iota(jnp.int32, 16)
    zeros16 = jnp.zeros((16,), jnp.int32)
    imaxv = jnp.full((16,), IMAX, jnp.int32)
    INTMIN = jnp.int32(-IMAX - 1)

    def _scal(v):
        return v[0]

    def row_body(rr, carry):
        r = wid * RPW + rr
        pltpu.make_async_copy(ord_hbm.at[r], srow, sem3).wait()

        def ga(g, accs):
            base = g * 96
            return tuple(jnp.minimum(accs[j], srow[pl.ds(base + j * 16, 16)])
                         for j in range(6))
        accs = lax.fori_loop(0, 522, ga, (imaxv,) * 6)
        for j in range(4):
            v = srow[pl.ds(522 * 96 + j * 16, 16)]
            accs = tuple(jnp.minimum(accs[i], v) if i == j else accs[i]
                         for i in range(6))
        m = accs[0]
        mn = accs[0]
        for j in range(1, 6):
            m = jnp.maximum(m, accs[j])
            mn = jnp.minimum(mn, accs[j])
        U = m[0]
        LO = mn[0]
        for l in range(1, 16):
            U = jnp.maximum(U, m[l])
            LO = jnp.minimum(LO, mn[l])

        def gb(i, c):
            base = i * 64
            vs = [srow[pl.ds(base + u * 16, 16)] for u in range(4)]
            msks = [v <= U for v in vs]
            incls = [plsc.cumsum(jnp.where(mk, 1, 0)) for mk in msks]
            cm1 = jnp.minimum(c, CAP - 65) - 1
            for u in range(4):
                pos = cm1 + incls[u]
                plsc.store_scatter(cord, [pos], vs[u], mask=msks[u])
                plsc.store_scatter(cidx, [pos], base + u * 16 + iota,
                                   mask=msks[u])
                cm1 = cm1 + plsc.all_reduce_population_count(msks[u])
            return cm1 + 1
        c = lax.fori_loop(0, NV // 4, gb, zeros16)

        @pl.when(rr < RPW - 1)
        def _prefetch_next():
            pltpu.async_copy(ord_hbm.at[r + 1], srow, sem3)

        cnt = jnp.minimum(_scal(c), CAP)
        plsc.store_scatter(cord, [jnp.minimum(cnt + iota, CAP - 1)], imaxv,
                           mask=cnt + iota < CAP)
        plsc.store_scatter(cord, [jnp.minimum(cnt + 16 + iota, CAP - 1)],
                           imaxv, mask=cnt + 16 + iota < CAP)
        nv2 = lax.shift_right_arithmetic(cnt + 31, 5)

        def count_le(t):
            def cb(j2, a):
                a = a + plsc.all_reduce_population_count(
                    cord[pl.ds(j2 * 32, 16)] <= t)
                return a + plsc.all_reduce_population_count(
                    cord[pl.ds(j2 * 32 + 16, 16)] <= t)
            return _scal(lax.fori_loop(0, nv2, cb, zeros16))

        def bs(lh):
            lo, hi = lh
            mid = lo + lax.shift_right_arithmetic(hi - lo, 1)
            ok = count_le(mid) >= CTX
            return (jnp.where(ok, lo, mid + 1), jnp.where(ok, mid, hi))
        _, K = lax.while_loop(lambda lh: lh[0] < lh[1], bs, (LO, U))
        c_lt = jnp.where(K == INTMIN, 0, count_le(K - 1))
        R = CTX - c_lt

        def count_eq_lt(t):
            def cb(j2, a):
                o = cord[pl.ds(j2 * 32, 16)]
                o2 = cord[pl.ds(j2 * 32 + 16, 16)]
                idv = cidx[pl.ds(j2 * 32, 16)]
                idv2 = cidx[pl.ds(j2 * 32 + 16, 16)]
                a = a + plsc.all_reduce_population_count((o == K) & (idv < t))
                return a + plsc.all_reduce_population_count(
                    (o2 == K) & (idv2 < t))
            return _scal(lax.fori_loop(0, nv2, cb, zeros16))

        def bsi(lh):
            lo, hi = lh
            mid = lo + lax.shift_right_arithmetic(hi - lo, 1)
            ok = count_eq_lt(mid) >= R
            return (jnp.where(ok, lo, mid + 1), jnp.where(ok, mid, hi))
        _, I = lax.while_loop(lambda lh: lh[0] < lh[1], bsi,
                              (jnp.int32(0), jnp.int32(NPAD)))
        nv = lax.shift_right_arithmetic(cnt + 15, 4)

        def ge(j2, c2):
            o = cord[pl.ds(j2 * 16, 16)]
            idv = cidx[pl.ds(j2 * 16, 16)]
            take = (o < K) | ((o == K) & (idv < I))
            mi = take.astype(jnp.int32)
            excl = plsc.cumsum(mi) - mi
            pos = jnp.minimum(c2 + excl, CTX - 1)
            plsc.store_scatter(oidx, [pos], idv, mask=take)
            return c2 + plsc.all_reduce_population_count(take)
        lax.fori_loop(0, nv, ge, zeros16)

        @pl.when(rr > 0)
        def _drain_prev():
            pltpu.make_async_copy(ctxbuf, ctx_hbm.at[r - 1], sem2).wait()

        pltpu.async_copy(bank_hbm.at[oidx], ctxbuf, sem).wait()
        pltpu.async_copy(ctxbuf, ctx_hbm.at[r], sem2)
        return carry

    pltpu.async_copy(ord_hbm.at[wid * RPW], srow, sem3)
    lax.fori_loop(0, RPW, row_body, 0)
    pltpu.make_async_copy(ctxbuf, ctx_hbm.at[wid * RPW + RPW - 1], sem2).wait()


def _ctx_body(x_ref, k_ref, ctx_ref, wlab, blab, Wt1, bt1, Wt2,
              plw, plb, Wp1, bp1, Wp2, bp2, gh, bh, Wh, bhd, out_ref):
    ck = ctx_ref[:, :, :D]
    yc = ctx_ref[:, :, D:D + 1]
    kq = k_ref[...]
    sim = 2.0 * jnp.sum(kq[:, None, :] * ck, axis=-1) - jnp.sum(ck * ck, axis=-1)
    m = jnp.max(sim, axis=-1, keepdims=True)
    p = jnp.exp(sim - m)
    p = p / jnp.sum(p, axis=-1, keepdims=True)

    diff = (kq[:, None, :] - ck).reshape(QB * CTX, D)
    t = _mm(jax.nn.relu(_mm(diff, Wt1[...]) + bt1[...]), Wt2[...])
    vals = yc * wlab[...][None] + blab[...][None] + t.reshape(QB, CTX, D)
    ctx_x = jnp.sum(p[:, :, None] * vals, axis=1)

    x2 = x_ref[...] + ctx_x
    X = jnp.broadcast_to(x2[:, None, :], (QB, KE, D))
    mu = jnp.mean(X, axis=-1, keepdims=True)
    var = jnp.mean((X - mu) ** 2, axis=-1, keepdims=True)
    h = (X - mu) / jnp.sqrt(var + EPS) * plw[...][None] + plb[...][None]
    h2 = h.reshape(QB * KE, D)
    h2 = _mm(jax.nn.relu(_mm(h2, Wp1[...]) + bp1[...]), Wp2[...]) + bp2[...]
    X = X + h2.reshape(QB, KE, D)
    o = _ln(X, gh[...][None], bh[...][None])
    o = jax.nn.relu(o).reshape(QB * KE, D)
    out_ref[...] = _mm(o, Wh[...]) + bhd[...]


def _full(shape):
    nd = len(shape)
    return pl.BlockSpec(shape, lambda *_: (0,) * nd)


def kernel(x_num, candidate_x_num, candidate_y, W_lin, b_lin, W_e1, b_e1,
           W_e2, b_e2, g_mix, b_mix, W_K, b_K, w_label, b_label, W_t1, b_t1,
           W_t2, pln_w, pln_b, W_p1, b_p1, W_p2, b_p2, g_hln, b_hln, W_head,
           b_head, is_train):
    f32 = jnp.float32
    cxp = jnp.zeros((NPAD, DIN), f32).at[:N].set(candidate_x_num)
    y128 = jnp.broadcast_to(
        jnp.zeros((NPAD,), f32).at[:N].set(candidate_y)[:, None], (NPAD, DIN))
    bl = b_lin.reshape(1, D)
    b1 = b_e1.reshape(1, DB)
    b2 = b_e2.reshape(1, D)
    g = g_mix.reshape(1, D)
    bm = b_mix.reshape(1, D)
    bK = b_K.reshape(1, D)

    wspecs = [_full(W_lin.shape), _full((1, D)), _full(W_e1.shape),
              _full((1, DB)), _full(W_e2.shape), _full((1, D)),
              _full((1, D)), _full((1, D)), _full(W_K.shape), _full((1, D))]
    wargs = (W_lin, bl, W_e1, b1, W_e2, b2, g, bm, W_K, bK)

    bank = pl.pallas_call(
        _bank_body,
        grid=(NBLK,),
        in_specs=[pl.BlockSpec((RB, DIN), lambda j: (j, 0)),
                  pl.BlockSpec((RB, DIN), lambda j: (j, 0))] + wspecs,
        out_specs=pl.BlockSpec((RB, BANKW), lambda j: (j, 0)),
        out_shape=jax.ShapeDtypeStruct((NPAD, BANKW), f32),
    )(cxp, y128, *wargs)

    xq, kq = pl.pallas_call(
        _query_body,
        in_specs=[_full((B, DIN))] + wspecs,
        out_specs=[_full((B, D)), _full((B, D))],
        out_shape=[jax.ShapeDtypeStruct((B, D), f32),
                   jax.ShapeDtypeStruct((B, D), f32)],
    )(x_num, *wargs)

    NS = 2
    HB = B // NS
    outs = []
    for h in range(NS):
        kq_h = kq[h * HB:(h + 1) * HB]
        xq_h = xq[h * HB:(h + 1) * HB]
        ordm = pl.pallas_call(
            _dist_body,
            grid=(NBLK,),
            in_specs=[_full((HB, D)),
                      pl.BlockSpec((RB, BANKW), lambda j: (j, 0))],
            out_specs=pl.BlockSpec((HB, RB), lambda j: (0, j)),
            out_shape=jax.ShapeDtypeStruct((HB, NPAD), jnp.int32),
        )(kq_h, bank)

        ctx = pl.kernel(
            _make_sc_body(HB // NW),
            out_type=jax.ShapeDtypeStruct((HB, CTX, BANKW), f32),
            mesh=plsc.VectorSubcoreMesh(core_axis_name="c",
                                        subcore_axis_name="s"),
            compiler_params=pltpu.CompilerParams(needs_layout_passes=False),
            scratch_types=[
                pltpu.VMEM((NPAD,), jnp.int32),
                pltpu.VMEM((CAP,), jnp.int32),
                pltpu.VMEM((CAP,), jnp.int32),
                pltpu.VMEM((CTX,), jnp.int32),
                pltpu.VMEM((CTX, BANKW), f32),
                pltpu.SemaphoreType.DMA,
                pltpu.SemaphoreType.DMA,
                pltpu.SemaphoreType.DMA,
            ],
        )(ordm, bank)

        out_h = pl.pallas_call(
            _ctx_body,
            grid=(HB // QB,),
            in_specs=[pl.BlockSpec((QB, D), lambda j: (j, 0)),
                      pl.BlockSpec((QB, D), lambda j: (j, 0)),
                      pl.BlockSpec((QB, CTX, BANKW), lambda j: (j, 0, 0)),
                      _full((1, D)), _full((1, D)), _full(W_t1.shape),
                      _full((1, DB)), _full(W_t2.shape), _full(pln_w.shape),
                      _full(pln_b.shape), _full(W_p1.shape), _full((1, DB)),
                      _full(W_p2.shape), _full((1, D)), _full((1, D)),
                      _full((1, D)), _full(W_head.shape), _full((1, 1))],
            out_specs=pl.BlockSpec((QB * KE, 1), lambda j: (j, 0)),
            out_shape=jax.ShapeDtypeStruct((HB * KE, 1), f32),
        )(xq_h, kq_h, ctx, w_label.reshape(1, D), b_label.reshape(1, D), W_t1,
          b_t1.reshape(1, DB), W_t2, pln_w, pln_b, W_p1, b_p1.reshape(1, DB),
          W_p2, b_p2.reshape(1, D), g_hln.reshape(1, D), b_hln.reshape(1, D),
          W_head, b_head.reshape(1, 1))
        outs.append(out_h.reshape(HB, KE, 1))

    return jnp.concatenate(outs, axis=0)


# speedup vs baseline: 11.8431x; 1.0359x over previous
import functools

import jax
import jax.numpy as jnp
from jax import lax
from jax.experimental import pallas as pl
from jax.experimental.pallas import tpu as pltpu
from jax.experimental.pallas import tpu_sc as plsc

B = 1024
N = 50000
NPAD = 50176
DIN = 128
D = 256
DB = 512
CTX = 96
KE = 4
EPS = 1e-5
RB = 1024
NBLK = NPAD // RB
QB = 16
BANKW = D + DIN
IMAX = 2147483647
PADORD = 0x7F000000

NW = 32
RPW = B // NW
NV = NPAD // 16
CAP = 8192
HIGHEST = lax.Precision.DEFAULT


def _ln(x, g, b):
    mu = jnp.mean(x, axis=-1, keepdims=True)
    var = jnp.mean((x - mu) ** 2, axis=-1, keepdims=True)
    return (x - mu) / jnp.sqrt(var + EPS) * g + b


def _mm(a, b):
    return lax.dot_general(a, b, (((1,), (0,)), ((), ())),
                           preferred_element_type=jnp.float32,
                           precision=HIGHEST)


def _encode(cx, Wl, bl, W1, b1, W2, b2, g, bm, WK, bK):
    x = _mm(cx, Wl) + bl
    h = _mm(jax.nn.relu(_mm(x, W1) + b1), W2) + b2
    x = x + h
    k = _mm(_ln(x, g, bm), WK) + bK
    return x, k


def _bank_body(cx_ref, y_ref, Wl, bl, W1, b1, W2, b2, g, bm, WK, bK, out_ref):
    _, ck = _encode(cx_ref[...], Wl[...], bl[...], W1[...], b1[...],
                    W2[...], b2[...], g[...], bm[...], WK[...], bK[...])
    out_ref[:, :D] = ck
    out_ref[:, D:] = y_ref[...]


def _query_body(x_ref, Wl, bl, W1, b1, W2, b2, g, bm, WK, bK, x_out, k_out):
    x, k = _encode(x_ref[...], Wl[...], bl[...], W1[...], b1[...],
                   W2[...], b2[...], g[...], bm[...], WK[...], bK[...])
    x_out[...] = x
    k_out[...] = k


def _dist_body(k_ref, bank_ref, o_ref):
    j = pl.program_id(0)
    ck = bank_ref[:, :D]
    kq = k_ref[...]
    dot = lax.dot_general(kq, ck, (((1,), (1,)), ((), ())),
                          preferred_element_type=jnp.float32,
                          precision=HIGHEST)
    cn = lax.dot_general(jnp.ones((1, D), jnp.float32), ck * ck,
                         (((1,), (1,)), ((), ())),
                         preferred_element_type=jnp.float32,
                         precision=lax.Precision.HIGHEST)
    kn = jnp.sum(kq * kq, axis=1, keepdims=True)
    s = (kn - 2.0 * dot) + cn
    bits = lax.bitcast_convert_type(s, jnp.int32)
    o = jnp.where(bits < 0, bits ^ 0x7FFFFFFF, bits)
    col = j * RB + lax.broadcasted_iota(jnp.int32, (1, RB), 1)
    o_ref[...] = jnp.where(col >= N, PADORD, o)


def _make_sc_body(rpw):
    def _sc_body(ord_hbm, bank_hbm, ctx_hbm, srow, cord, cidx, oidx, ctxbuf,
                 sem, sem2, sem3):
        return _sc_rows(rpw, ord_hbm, bank_hbm, ctx_hbm, srow, cord, cidx,
                        oidx, ctxbuf, sem, sem2, sem3)
    return _sc_body


def _sc_rows(RPW, ord_hbm, bank_hbm, ctx_hbm, srow, cord, cidx, oidx, ctxbuf,
             sem, sem2, sem3):
    wid = lax.axis_index("s") * 2 + lax.axis_index("c")
    iota = lax.iota(jnp.int32, 16)
    zeros16 = jnp.zeros((16,), jnp.int32)
    imaxv = jnp.full((16,), IMAX, jnp.int32)
    INTMIN = jnp.int32(-IMAX - 1)

    def _scal(v):
        return v[0]

    def row_body(rr, carry):
        r = wid * RPW + rr
        pltpu.make_async_copy(ord_hbm.at[r], srow, sem3).wait()

        def ga(g, accs):
            base = g * 96
            return tuple(jnp.minimum(accs[j], srow[pl.ds(base + j * 16, 16)])
                         for j in range(6))
        accs = lax.fori_loop(0, 522, ga, (imaxv,) * 6)
        for j in range(4):
            v = srow[pl.ds(522 * 96 + j * 16, 16)]
            accs = tuple(jnp.minimum(accs[i], v) if i == j else accs[i]
                         for i in range(6))
        m = accs[0]
        mn = accs[0]
        for j in range(1, 6):
            m = jnp.maximum(m, accs[j])
            mn = jnp.minimum(mn, accs[j])
        U = m[0]
        LO = mn[0]
        for l in range(1, 16):
            U = jnp.maximum(U, m[l])
            LO = jnp.minimum(LO, mn[l])

        def gb(i, c):
            base = i * 64
            vs = [srow[pl.ds(base + u * 16, 16)] for u in range(4)]
            msks = [v <= U for v in vs]
            incls = [plsc.cumsum(jnp.where(mk, 1, 0)) for mk in msks]
            cm1 = jnp.minimum(c, CAP - 65) - 1
            for u in range(4):
                pos = cm1 + incls[u]
                plsc.store_scatter(cord, [pos], vs[u], mask=msks[u])
                plsc.store_scatter(cidx, [pos], base + u * 16 + iota,
                                   mask=msks[u])
                cm1 = cm1 + plsc.all_reduce_population_count(msks[u])
            return cm1 + 1
        c = lax.fori_loop(0, NV // 4, gb, zeros16)

        @pl.when(rr < RPW - 1)
        def _prefetch_next():
            pltpu.async_copy(ord_hbm.at[r + 1], srow, sem3)

        cnt = jnp.minimum(_scal(c), CAP)
        plsc.store_scatter(cord, [jnp.minimum(cnt + iota, CAP - 1)], imaxv,
                           mask=cnt + iota < CAP)
        plsc.store_scatter(cord, [jnp.minimum(cnt + 16 + iota, CAP - 1)],
                           imaxv, mask=cnt + 16 + iota < CAP)
        nv2 = lax.shift_right_arithmetic(cnt + 31, 5)

        def count_le(t):
            def cb(j2, a):
                a = a + plsc.all_reduce_population_count(
                    cord[pl.ds(j2 * 32, 16)] <= t)
                return a + plsc.all_reduce_population_count(
                    cord[pl.ds(j2 * 32 + 16, 16)] <= t)
            return _scal(lax.fori_loop(0, nv2, cb, zeros16))

        def bs(lh):
            lo, hi = lh
            mid = lo + lax.shift_right_arithmetic(hi - lo, 1)
            ok = count_le(mid) >= CTX
            return (jnp.where(ok, lo, mid + 1), jnp.where(ok, mid, hi))
        _, K = lax.while_loop(lambda lh: lh[0] < lh[1], bs, (LO, U))
        c_lt = jnp.where(K == INTMIN, 0, count_le(K - 1))
        R = CTX - c_lt

        def count_eq_lt(t):
            def cb(j2, a):
                o = cord[pl.ds(j2 * 32, 16)]
                o2 = cord[pl.ds(j2 * 32 + 16, 16)]
                idv = cidx[pl.ds(j2 * 32, 16)]
                idv2 = cidx[pl.ds(j2 * 32 + 16, 16)]
                a = a + plsc.all_reduce_population_count((o == K) & (idv < t))
                return a + plsc.all_reduce_population_count(
                    (o2 == K) & (idv2 < t))
            return _scal(lax.fori_loop(0, nv2, cb, zeros16))

        def bsi(lh):
            lo, hi = lh
            mid = lo + lax.shift_right_arithmetic(hi - lo, 1)
            ok = count_eq_lt(mid) >= R
            return (jnp.where(ok, lo, mid + 1), jnp.where(ok, mid, hi))
        _, I = lax.while_loop(lambda lh: lh[0] < lh[1], bsi,
                              (jnp.int32(0), jnp.int32(NPAD)))
        nv = lax.shift_right_arithmetic(cnt + 15, 4)

        def ge(j2, c2):
            o = cord[pl.ds(j2 * 16, 16)]
            idv = cidx[pl.ds(j2 * 16, 16)]
            take = (o < K) | ((o == K) & (idv < I))
            mi = take.astype(jnp.int32)
            excl = plsc.cumsum(mi) - mi
            pos = jnp.minimum(c2 + excl, CTX - 1)
            plsc.store_scatter(oidx, [pos], idv, mask=take)
            return c2 + plsc.all_reduce_population_count(take)
        lax.fori_loop(0, nv, ge, zeros16)

        @pl.when(rr > 0)
        def _drain_prev():
            pltpu.make_async_copy(ctxbuf, ctx_hbm.at[r - 1], sem2).wait()

        pltpu.async_copy(bank_hbm.at[oidx], ctxbuf, sem).wait()
        pltpu.async_copy(ctxbuf, ctx_hbm.at[r], sem2)
        return carry

    pltpu.async_copy(ord_hbm.at[wid * RPW], srow, sem3)
    lax.fori_loop(0, RPW, row_body, 0)
    pltpu.make_async_copy(ctxbuf, ctx_hbm.at[wid * RPW + RPW - 1], sem2).wait()


def _ctx_body(x_ref, k_ref, ctx_ref, wlab, blab, Wt1, bt1, Wt2,
              plw, plb, Wp1, bp1, Wp2, bp2, gh, bh, Wh, bhd, out_ref):
    ck = ctx_ref[:, :, :D]
    yc = ctx_ref[:, :, D:D + 1]
    kq = k_ref[...]
    sim = 2.0 * jnp.sum(kq[:, None, :] * ck, axis=-1) - jnp.sum(ck * ck, axis=-1)
    m = jnp.max(sim, axis=-1, keepdims=True)
    p = jnp.exp(sim - m)
    p = p / jnp.sum(p, axis=-1, keepdims=True)

    diff = (kq[:, None, :] - ck).reshape(QB * CTX, D)
    t = _mm(jax.nn.relu(_mm(diff, Wt1[...]) + bt1[...]), Wt2[...])
    vals = yc * wlab[...][None] + blab[...][None] + t.reshape(QB, CTX, D)
    ctx_x = jnp.sum(p[:, :, None] * vals, axis=1)

    x2 = x_ref[...] + ctx_x
    X = jnp.broadcast_to(x2[:, None, :], (QB, KE, D))
    mu = jnp.mean(X, axis=-1, keepdims=True)
    var = jnp.mean((X - mu) ** 2, axis=-1, keepdims=True)
    h = (X - mu) / jnp.sqrt(var + EPS) * plw[...][None] + plb[...][None]
    h2 = h.reshape(QB * KE, D)
    h2 = _mm(jax.nn.relu(_mm(h2, Wp1[...]) + bp1[...]), Wp2[...]) + bp2[...]
    X = X + h2.reshape(QB, KE, D)
    o = _ln(X, gh[...][None], bh[...][None])
    o = jax.nn.relu(o).reshape(QB * KE, D)
    out_ref[...] = _mm(o, Wh[...]) + bhd[...]


def _full(shape):
    nd = len(shape)
    return pl.BlockSpec(shape, lambda *_: (0,) * nd)


def kernel(x_num, candidate_x_num, candidate_y, W_lin, b_lin, W_e1, b_e1,
           W_e2, b_e2, g_mix, b_mix, W_K, b_K, w_label, b_label, W_t1, b_t1,
           W_t2, pln_w, pln_b, W_p1, b_p1, W_p2, b_p2, g_hln, b_hln, W_head,
           b_head, is_train):
    f32 = jnp.float32
    cxp = jnp.zeros((NPAD, DIN), f32).at[:N].set(candidate_x_num)
    y128 = jnp.broadcast_to(
        jnp.zeros((NPAD,), f32).at[:N].set(candidate_y)[:, None], (NPAD, DIN))
    bl = b_lin.reshape(1, D)
    b1 = b_e1.reshape(1, DB)
    b2 = b_e2.reshape(1, D)
    g = g_mix.reshape(1, D)
    bm = b_mix.reshape(1, D)
    bK = b_K.reshape(1, D)

    wspecs = [_full(W_lin.shape), _full((1, D)), _full(W_e1.shape),
              _full((1, DB)), _full(W_e2.shape), _full((1, D)),
              _full((1, D)), _full((1, D)), _full(W_K.shape), _full((1, D))]
    wargs = (W_lin, bl, W_e1, b1, W_e2, b2, g, bm, W_K, bK)

    bank = pl.pallas_call(
        _bank_body,
        grid=(NBLK,),
        in_specs=[pl.BlockSpec((RB, DIN), lambda j: (j, 0)),
                  pl.BlockSpec((RB, DIN), lambda j: (j, 0))] + wspecs,
        out_specs=pl.BlockSpec((RB, BANKW), lambda j: (j, 0)),
        out_shape=jax.ShapeDtypeStruct((NPAD, BANKW), f32),
    )(cxp, y128, *wargs)

    xq, kq = pl.pallas_call(
        _query_body,
        in_specs=[_full((B, DIN))] + wspecs,
        out_specs=[_full((B, D)), _full((B, D))],
        out_shape=[jax.ShapeDtypeStruct((B, D), f32),
                   jax.ShapeDtypeStruct((B, D), f32)],
    )(x_num, *wargs)

    NS = 4
    HB = B // NS
    outs = []
    for h in range(NS):
        kq_h = kq[h * HB:(h + 1) * HB]
        xq_h = xq[h * HB:(h + 1) * HB]
        ordm = pl.pallas_call(
            _dist_body,
            grid=(NBLK,),
            in_specs=[_full((HB, D)),
                      pl.BlockSpec((RB, BANKW), lambda j: (j, 0))],
            out_specs=pl.BlockSpec((HB, RB), lambda j: (0, j)),
            out_shape=jax.ShapeDtypeStruct((HB, NPAD), jnp.int32),
        )(kq_h, bank)

        ctx = pl.kernel(
            _make_sc_body(HB // NW),
            out_type=jax.ShapeDtypeStruct((HB, CTX, BANKW), f32),
            mesh=plsc.VectorSubcoreMesh(core_axis_name="c",
                                        subcore_axis_name="s"),
            compiler_params=pltpu.CompilerParams(needs_layout_passes=False),
            scratch_types=[
                pltpu.VMEM((NPAD,), jnp.int32),
                pltpu.VMEM((CAP,), jnp.int32),
                pltpu.VMEM((CAP,), jnp.int32),
                pltpu.VMEM((CTX,), jnp.int32),
                pltpu.VMEM((CTX, BANKW), f32),
                pltpu.SemaphoreType.DMA,
                pltpu.SemaphoreType.DMA,
                pltpu.SemaphoreType.DMA,
            ],
        )(ordm, bank)

        out_h = pl.pallas_call(
            _ctx_body,
            grid=(HB // QB,),
            in_specs=[pl.BlockSpec((QB, D), lambda j: (j, 0)),
                      pl.BlockSpec((QB, D), lambda j: (j, 0)),
                      pl.BlockSpec((QB, CTX, BANKW), lambda j: (j, 0, 0)),
                      _full((1, D)), _full((1, D)), _full(W_t1.shape),
                      _full((1, DB)), _full(W_t2.shape), _full(pln_w.shape),
                      _full(pln_b.shape), _full(W_p1.shape), _full((1, DB)),
                      _full(W_p2.shape), _full((1, D)), _full((1, D)),
                      _full((1, D)), _full(W_head.shape), _full((1, 1))],
            out_specs=pl.BlockSpec((QB * KE, 1), lambda j: (j, 0)),
            out_shape=jax.ShapeDtypeStruct((HB * KE, 1), f32),
        )(xq_h, kq_h, ctx, w_label.reshape(1, D), b_label.reshape(1, D), W_t1,
          b_t1.reshape(1, DB), W_t2, pln_w, pln_b, W_p1, b_p1.reshape(1, DB),
          W_p2, b_p2.reshape(1, D), g_hln.reshape(1, D), b_hln.reshape(1, D),
          W_head, b_head.reshape(1, 1))
        outs.append(out_h.reshape(HB, KE, 1))

    return jnp.concatenate(outs, axis=0)
